# baseline (speedup 1.0000x reference)
import functools

import numpy as np
import jax
import jax.numpy as jnp
from jax.experimental import pallas as pl
from jax.experimental.pallas import tpu as pltpu


def _matmul_pool_kernel(w_ref, a_ref, pool_ref, o_ref, acc_ref, *, nk, g):
    k = pl.program_id(1)

    @pl.when(k == 0)
    def _init():
        acc_ref[...] = jnp.zeros_like(acc_ref)

    acc_ref[...] += jnp.dot(w_ref[...], a_ref[...],
                            preferred_element_type=jnp.float32)

    @pl.when(k == nk - 1)
    def _epilogue():
        acc = acc_ref[...]
        mx = jnp.maximum(jnp.maximum(acc[:, 0 * g:1 * g], acc[:, 1 * g:2 * g]),
                         jnp.maximum(acc[:, 2 * g:3 * g], acc[:, 3 * g:4 * g]))
        y = jnp.where(mx >= 0.0, mx, 0.01 * mx)
        o_ref[...] = jnp.dot(y.astype(jnp.bfloat16), pool_ref[...],
                             preferred_element_type=jnp.float32)


@functools.partial(jax.jit, static_argnames=("kernel_size", "padding"))
def _forward(x, w, b, gamma, beta, running_mean, running_var,
             *, kernel_size=7, padding=2, eps=1e-5):
    N, Cin, H, W = x.shape
    Cout = w.shape[0]
    KH = KW = kernel_size

    OH = H + 2 * padding - KH + 1
    OW = W + 2 * padding - KW + 1
    MH, MW = OH // 2, OW // 2
    AH, AW = MH // 2, MW // 2
    MHn, MWn = 2 * AH, 2 * AW
    K = KH * KW * Cin
    G = N * MHn * MWn
    P = N * AH * AW

    NK = 4
    KC = 3200
    Kp = NK * KC
    CB = Cout // 2

    scale = gamma * jax.lax.rsqrt(running_var + eps)
    shift = beta + scale * (b - running_mean)

    wt = (jnp.transpose(w, (0, 2, 3, 1)) * scale[:, None, None, None])
    wt = wt.reshape(Cout, K)
    wt = jnp.concatenate([wt, shift[:, None]], axis=1)
    wt = jnp.pad(wt, ((0, 0), (0, Kp - (K + 1)))).astype(jnp.bfloat16)

    xp = jnp.pad(x, ((0, 0), (0, 0), (padding, padding), (padding, padding)))
    xp = jnp.transpose(xp, (1, 0, 2, 3)).astype(jnp.bfloat16)
    blocks = []
    for kh in range(KH):
        for kw in range(KW):
            offs = [xp[:, :, di + kh:di + kh + 2 * MHn - 1:2,
                        dj + kw:dj + kw + 2 * MWn - 1:2]
                    for di in range(2) for dj in range(2)]
            blocks.append(jnp.stack(offs, axis=1).reshape(Cin, 4 * G))
    a = jnp.concatenate(blocks, axis=0)
    a = jnp.concatenate([a, jnp.ones((1, 4 * G), jnp.bfloat16)], axis=0)
    a = jnp.pad(a, ((0, Kp - (K + 1)), (0, 0)))

    pool = np.zeros((G, 128), np.float32)
    for n in range(N):
        for mh in range(MHn):
            for mw in range(MWn):
                src = n * MHn * MWn + mh * MWn + mw
                dst = n * AH * AW + (mh // 2) * AW + (mw // 2)
                pool[src, dst] = 0.25
    pool = jnp.asarray(pool, dtype=jnp.bfloat16)

    flops = 2 * Cout * Kp * (4 * G) + 2 * Cout * G * 128
    bytes_accessed = 2 * (Cout * Kp + Kp * 4 * G + G * 128) + 4 * Cout * 128
    out2d = pl.pallas_call(
        functools.partial(_matmul_pool_kernel, nk=NK, g=G),
        out_shape=jax.ShapeDtypeStruct((Cout, 128), jnp.float32),
        grid=(2, NK),
        in_specs=[
            pl.BlockSpec((CB, KC), lambda i, k: (i, k)),
            pl.BlockSpec((KC, 4 * G), lambda i, k: (k, 0)),
            pl.BlockSpec((G, 128), lambda i, k: (0, 0)),
        ],
        out_specs=pl.BlockSpec((CB, 128), lambda i, k: (i, 0)),
        scratch_shapes=[pltpu.VMEM((CB, 4 * G), jnp.float32)],
        compiler_params=pltpu.CompilerParams(
            dimension_semantics=("parallel", "arbitrary")),
        cost_estimate=pl.CostEstimate(flops=flops, transcendentals=0,
                                      bytes_accessed=bytes_accessed),
    )(wt, a, pool)

    return jnp.transpose(out2d[:, :P].reshape(Cout, N, AH, AW), (1, 0, 2, 3))


def kernel(x, w, b, gamma, beta, running_mean, running_var):
    return _forward(x, w, b, gamma, beta, running_mean, running_var,
                    kernel_size=7, padding=2)


# speedup vs baseline: 5.1420x; 5.1420x over previous
import functools

import numpy as np
import jax
import jax.numpy as jnp
from jax.experimental import pallas as pl
from jax.experimental.pallas import tpu as pltpu


def _conv_pool_kernel(w_ref, a_ref, shift_ref, pool_ref, o_ref, *, kw_taps, kc, lw, lp):
    acc = jnp.zeros((w_ref.shape[0], lw), jnp.float32)
    for kw in range(kw_taps):
        acc += jnp.dot(w_ref[:, kw * kc:(kw + 1) * kc], a_ref[:, kw:kw + lw],
                       preferred_element_type=jnp.float32)
    m = jnp.maximum(jnp.maximum(acc[:, 0:lw - 21], acc[:, 1:lw - 20]),
                    jnp.maximum(acc[:, 20:lw - 1], acc[:, 21:lw]))
    m = m + shift_ref[:, 0:1]
    y = jnp.where(m >= 0.0, m, 0.01 * m)
    y = jnp.pad(y.astype(jnp.bfloat16), ((0, 0), (0, lw - (lw - 21))))
    o_ref[...] = jnp.dot(y, pool_ref[...], preferred_element_type=jnp.float32)


@functools.partial(jax.jit, static_argnames=("kernel_size", "padding"))
def _forward(x, w, b, gamma, beta, running_mean, running_var,
             *, kernel_size=7, padding=2, eps=1e-5):
    N, Cin, H, W = x.shape
    Cout = w.shape[0]
    KH = KW = kernel_size
    Hp = H + 2 * padding
    Wp = W + 2 * padding
    OHn = OWn = 12
    AH = AW = 3
    P = N * AH * AW
    KC = KH * Cin
    LW = N * OHn * Wp
    LP = 1024
    CB = Cout // 2

    scale = gamma * jax.lax.rsqrt(running_var + eps)
    shift = (beta + scale * (b - running_mean)).astype(jnp.float32)
    shift_col = jnp.broadcast_to(shift[:, None], (Cout, 128))

    wt = (jnp.transpose(w, (0, 3, 2, 1)) * scale[:, None, None, None])
    wt = wt.reshape(Cout, KW * KC).astype(jnp.bfloat16)

    xp = jnp.pad(x, ((0, 0), (0, 0), (padding, padding), (padding, padding)))
    xp = jnp.transpose(xp, (1, 0, 2, 3)).astype(jnp.bfloat16)
    a2 = jnp.stack([xp[:, :, kh:kh + OHn, :] for kh in range(KH)], axis=0)
    a2 = a2.reshape(KH * Cin, LW)
    a2 = jnp.pad(a2, ((0, 0), (0, LP - LW)))

    pool = np.zeros((LW, 128), np.float32)
    for n in range(N):
        for ah in range(AH):
            for aw in range(AW):
                dst = n * AH * AW + ah * AW + aw
                for da in range(2):
                    for db in range(2):
                        src = n * OHn * Wp + 2 * (2 * ah + da) * Wp + 2 * (2 * aw + db)
                        pool[src, dst] = 0.25
    pool = jnp.asarray(pool, dtype=jnp.bfloat16)

    flops = 2 * Cout * KW * KC * LW + 2 * Cout * LW * 128
    bytes_accessed = 2 * (Cout * KW * KC + KC * LP + LW * 128) + 4 * Cout * 256
    out2d = pl.pallas_call(
        functools.partial(_conv_pool_kernel, kw_taps=KW, kc=KC, lw=LW, lp=LP),
        out_shape=jax.ShapeDtypeStruct((Cout, 128), jnp.float32),
        grid=(2,),
        in_specs=[
            pl.BlockSpec((CB, KW * KC), lambda i: (i, 0)),
            pl.BlockSpec((KC, LP), lambda i: (0, 0)),
            pl.BlockSpec((CB, 128), lambda i: (i, 0)),
            pl.BlockSpec((LW, 128), lambda i: (0, 0)),
        ],
        out_specs=pl.BlockSpec((CB, 128), lambda i: (i, 0)),
        compiler_params=pltpu.CompilerParams(
            dimension_semantics=("parallel",)),
        cost_estimate=pl.CostEstimate(flops=flops, transcendentals=0,
                                      bytes_accessed=bytes_accessed),
    )(wt, a2, shift_col, pool)

    return jnp.transpose(out2d[:, :P].reshape(Cout, N, AH, AW), (1, 0, 2, 3))


def kernel(x, w, b, gamma, beta, running_mean, running_var):
    return _forward(x, w, b, gamma, beta, running_mean, running_var,
                    kernel_size=7, padding=2)


# speedup vs baseline: 7.0065x; 1.3626x over previous
import functools

import numpy as np
import jax
import jax.numpy as jnp
from jax.experimental import pallas as pl
from jax.experimental.pallas import tpu as pltpu


def _conv_pool_kernel(w_ref, x_ref, scale_ref, shift_ref, pool_ref, o_ref,
                      a2_ref, *, n_im, kh_taps, kw_taps, cin, wp, ohn, lw):
    nw = ohn * wp
    a2_ref[...] = jnp.zeros_like(a2_ref)
    for kh in range(kh_taps):
        for n in range(n_im):
            a2_ref[kh * cin:(kh + 1) * cin, n * nw:(n + 1) * nw] = (
                x_ref[n, :, kh * wp:kh * wp + nw])

    kc = kh_taps * cin
    acc = jnp.dot(w_ref[:, 0:kc], a2_ref[:, 0:lw],
                  preferred_element_type=jnp.float32)
    for kw in range(1, kw_taps):
        acc += jnp.dot(w_ref[:, kw * kc:(kw + 1) * kc], a2_ref[:, kw:kw + lw],
                       preferred_element_type=jnp.float32)

    acc = acc * scale_ref[:, 0:1]
    m = jnp.maximum(jnp.maximum(acc[:, 0:lw - 21], acc[:, 1:lw - 20]),
                    jnp.maximum(acc[:, 20:lw - 1], acc[:, 21:lw]))
    m = m + shift_ref[:, 0:1]
    y = jnp.where(m >= 0.0, m, 0.01 * m)
    y = jnp.pad(y.astype(jnp.bfloat16), ((0, 0), (0, 21)))
    o_ref[...] = jnp.dot(y, pool_ref[...], preferred_element_type=jnp.float32)


@functools.partial(jax.jit, static_argnames=("kernel_size", "padding"))
def _forward(x, w, b, gamma, beta, running_mean, running_var,
             *, kernel_size=7, padding=2, eps=1e-5):
    N, Cin, H, W = x.shape
    Cout = w.shape[0]
    KH = KW = kernel_size
    Hp, Wp = H + 2 * padding, W + 2 * padding
    OHn = 12
    AH = AW = 3
    P = N * AH * AW
    KC = KH * Cin
    LW = N * OHn * Wp
    CB = Cout // 2

    scale = (gamma * jax.lax.rsqrt(running_var + eps)).astype(jnp.float32)
    shift = (beta + scale * (b - running_mean)).astype(jnp.float32)
    scale_col = jnp.broadcast_to(scale[:, None], (Cout, 128))
    shift_col = jnp.broadcast_to(shift[:, None], (Cout, 128))

    wt = jnp.transpose(w, (0, 3, 2, 1)).reshape(Cout, KW * KC)
    wt = wt.astype(jnp.bfloat16)

    xp = jnp.pad(x, ((0, 0), (0, 0), (padding, padding), (padding, padding)))
    xp = xp.reshape(N, Cin, Hp * Wp).astype(jnp.bfloat16)
    xp = jnp.pad(xp, ((0, 0), (0, 0), (0, 512 - Hp * Wp)))

    pool = np.zeros((LW, 128), np.float32)
    for n in range(N):
        for ah in range(AH):
            for aw in range(AW):
                dst = n * AH * AW + ah * AW + aw
                for da in range(2):
                    for db in range(2):
                        src = n * OHn * Wp + 2 * (2 * ah + da) * Wp + 2 * (2 * aw + db)
                        pool[src, dst] = 0.25
    pool = jnp.asarray(pool, dtype=jnp.bfloat16)

    flops = 2 * Cout * KW * KC * LW + 2 * Cout * LW * 128
    bytes_accessed = 2 * (Cout * KW * KC + N * Cin * 512 + LW * 128) + 4 * Cout * 256
    out2d = pl.pallas_call(
        functools.partial(_conv_pool_kernel, n_im=N, kh_taps=KH, kw_taps=KW,
                          cin=Cin, wp=Wp, ohn=OHn, lw=LW),
        out_shape=jax.ShapeDtypeStruct((Cout, 128), jnp.float32),
        grid=(2,),
        in_specs=[
            pl.BlockSpec((CB, KW * KC), lambda i: (i, 0)),
            pl.BlockSpec((N, Cin, 512), lambda i: (0, 0, 0)),
            pl.BlockSpec((CB, 128), lambda i: (i, 0)),
            pl.BlockSpec((CB, 128), lambda i: (i, 0)),
            pl.BlockSpec((LW, 128), lambda i: (0, 0)),
        ],
        out_specs=pl.BlockSpec((CB, 128), lambda i: (i, 0)),
        scratch_shapes=[pltpu.VMEM((KC, 1024), jnp.bfloat16)],
        compiler_params=pltpu.CompilerParams(
            dimension_semantics=("parallel",)),
        cost_estimate=pl.CostEstimate(flops=flops, transcendentals=0,
                                      bytes_accessed=bytes_accessed),
    )(wt, xp, scale_col, shift_col, pool)

    return jnp.transpose(out2d[:, :P].reshape(Cout, N, AH, AW), (1, 0, 2, 3))


def kernel(x, w, b, gamma, beta, running_mean, running_var):
    return _forward(x, w, b, gamma, beta, running_mean, running_var,
                    kernel_size=7, padding=2)
